# baseline (speedup 1.0000x reference)
import functools

import jax
import jax.numpy as jnp
from jax import lax
from jax.experimental import pallas as pl
from jax.experimental.pallas import tpu as pltpu
from jax.experimental.pallas import tpu_sc as plsc

NUM_EMBEDDINGS = 1000000
EMBEDDING_DIM = 64
BT = 128
DT = 8


@functools.cache
def _build(nb: int, ns_seq: int):
    info = plsc.get_sparse_core_info()
    nc, nsub = info.num_cores, info.num_subcores
    nw = nc * nsub
    n_units = ns_seq * nb
    assert n_units % nw == 0
    per_worker = n_units // nw

    mesh = plsc.VectorSubcoreMesh(core_axis_name="c", subcore_axis_name="s")

    @functools.partial(
        pl.kernel,
        mesh=mesh,
        out_type=jax.ShapeDtypeStruct(
            (ns_seq, EMBEDDING_DIM // DT, nb, DT, BT), jnp.float32),
        scratch_types=[
            [pltpu.VMEM((BT,), jnp.int32)] * 2,
            [pltpu.VMEM((BT, EMBEDDING_DIM), jnp.float32)] * 2,
            [pltpu.VMEM((EMBEDDING_DIM // DT, DT, BT), jnp.float32)] * 2,
            [pltpu.SemaphoreType.DMA] * 2,
            [pltpu.SemaphoreType.DMA] * 2,
        ],
        compiler_params=pltpu.CompilerParams(
            use_tc_tiling_on_sc=False, needs_layout_passes=False),
    )
    def gather_kernel(w4_hbm, table_hbm, out_hbm,
                      idx_v, rows, planes, gsem, ssem):
        wid = lax.axis_index("s") * nc + lax.axis_index("c")
        u0 = wid * per_worker

        def unit_st(j):
            u = u0 + j
            return u // nb, u % nb

        def load_idx_and_gather(b, j):
            s, tj = unit_st(j)
            pltpu.sync_copy(w4_hbm.at[s // DT, tj, s % DT], idx_v[b])
            pltpu.async_copy(table_hbm.at[idx_v[b]], rows[b], gsem[b])

        def wait_gather(b, j):
            s, tj = unit_st(j)
            pltpu.make_async_copy(table_hbm.at[idx_v[b]], rows[b],
                                  gsem[b]).wait()

        def transpose(b):
            src = rows[b]
            dst = planes[b]
            lanes = lax.iota(jnp.int32, 16)

            @pl.loop(0, EMBEDDING_DIM)
            def _(d):
                col = jnp.full((16,), 0, jnp.int32) + d
                for bc in range(BT // 16):
                    row = lanes + (bc * 16)
                    vals = plsc.load_gather(src, [row, col])
                    dst[d // DT, d % DT, pl.ds(bc * 16, 16)] = vals

        def start_store(b, j):
            s, tj = unit_st(j)
            pltpu.async_copy(planes[b], out_hbm.at[s, :, tj], ssem[b])

        def wait_store(b, j):
            s, tj = unit_st(j)
            pltpu.make_async_copy(planes[b], out_hbm.at[s, :, tj],
                                  ssem[b]).wait()

        assert per_worker % 2 == 0
        load_idx_and_gather(0, 0)

        @pl.loop(0, per_worker // 2 - 1)
        def _(p):
            for b in range(2):
                j = 2 * p + b
                load_idx_and_gather(1 - b, j + 1)
                wait_gather(b, j)
                pl.when(p >= 1)(lambda: wait_store(b, j - 2))
                transpose(b)
                start_store(b, j)

        j = per_worker - 2
        load_idx_and_gather(1, j + 1)
        wait_gather(0, j)
        wait_store(0, j - 2)
        transpose(0)
        start_store(0, j)
        wait_gather(1, j + 1)
        wait_store(1, j - 1)
        transpose(1)
        start_store(1, j + 1)
        wait_store(0, j)
        wait_store(1, j + 1)

    return gather_kernel


def kernel(words, weight):
    nb4, ns_seq = words.shape
    nb = nb4 // BT
    w4 = (words.astype(jnp.int32)
          .reshape(nb, BT, ns_seq // DT, DT)
          .transpose(2, 0, 3, 1))
    out5 = _build(nb, ns_seq)(w4, weight)
    out = out5.transpose(2, 4, 0, 1, 3).reshape(nb4, ns_seq, EMBEDDING_DIM)
    return out


# speedup vs baseline: 1.2047x; 1.2047x over previous
import functools

import jax
import jax.numpy as jnp
from jax import lax
from jax.experimental import pallas as pl
from jax.experimental.pallas import tpu as pltpu
from jax.experimental.pallas import tpu_sc as plsc

NUM_EMBEDDINGS = 1000000
EMBEDDING_DIM = 64
BT = 128
DT = 8


@functools.cache
def _build(nb: int, ns_seq: int):
    info = plsc.get_sparse_core_info()
    nc, nsub = info.num_cores, info.num_subcores
    nw = nc * nsub
    n_units = ns_seq * nb
    assert n_units % nw == 0
    per_worker = n_units // nw
    assert per_worker % 2 == 0 and per_worker >= 4

    mesh = plsc.VectorSubcoreMesh(core_axis_name="c", subcore_axis_name="s")

    @functools.partial(
        pl.kernel,
        mesh=mesh,
        out_type=jax.ShapeDtypeStruct(
            (ns_seq, EMBEDDING_DIM // DT, nb, DT, BT), jnp.float32),
        scratch_types=[
            [pltpu.VMEM((BT,), jnp.int32)] * 2,
            [pltpu.VMEM((BT, EMBEDDING_DIM), jnp.float32)] * 2,
            [pltpu.VMEM((EMBEDDING_DIM // DT, DT, BT), jnp.float32)] * 2,
            [pltpu.SemaphoreType.DMA] * 2,
            [pltpu.SemaphoreType.DMA] * 2,
            [pltpu.SemaphoreType.DMA] * 2,
        ],
        compiler_params=pltpu.CompilerParams(
            use_tc_tiling_on_sc=False,
            needs_layout_passes=False,
            disable_bounds_checks=True,
        ),
    )
    def gather_kernel(w4_hbm, table_hbm, out_hbm,
                      idx_v, rows, planes, isem, gsem, ssem):
        wid = lax.axis_index("s") * nc + lax.axis_index("c")
        u0 = wid * per_worker

        def unit_st(j):
            u = u0 + j
            return u // nb, u % nb

        def idx_slice(j):
            s, tj = unit_st(j)
            return w4_hbm.at[s // DT, tj, s % DT]

        def start_idx(b, j):
            pltpu.async_copy(idx_slice(j), idx_v[b], isem[b])

        def wait_idx(b, j):
            pltpu.make_async_copy(idx_slice(j), idx_v[b], isem[b]).wait()

        def start_gather(b):
            pltpu.async_copy(table_hbm.at[idx_v[b]], rows[b], gsem[b])

        def wait_gather(b):
            pltpu.make_async_copy(table_hbm.at[idx_v[b]], rows[b],
                                  gsem[b]).wait()

        def out_slice(j):
            s, tj = unit_st(j)
            return out_hbm.at[s, :, tj]

        def start_store(b, j):
            pltpu.async_copy(planes[b], out_slice(j), ssem[b])

        def wait_store(b, j):
            pltpu.make_async_copy(planes[b], out_slice(j), ssem[b]).wait()

        lanes16 = lax.iota(jnp.int32, 16)
        div = lanes16 % DT
        tivs = [lanes16 // DT + 2 * dc for dc in range(EMBEDDING_DIM // 16)]

        def transpose(b):
            src = rows[b]
            dst = planes[b]

            @pl.loop(0, BT, unroll=4)
            def _(bi):
                bvec = jnp.zeros((16,), jnp.int32) + bi
                for dc in range(EMBEDDING_DIM // 16):
                    v = src[bi, pl.ds(dc * 16, 16)]
                    plsc.store_scatter(dst, [tivs[dc], div, bvec], v)

        start_idx(0, 0)
        wait_idx(0, 0)
        start_gather(0)
        start_idx(1, 1)

        @pl.loop(0, per_worker // 2 - 1)
        def _(p):
            for b in range(2):
                j = 2 * p + b
                b1 = 1 - b
                wait_idx(b1, j + 1)
                start_gather(b1)
                wait_gather(b)
                start_idx(b, j + 2)
                pl.when(p >= 1)(lambda: wait_store(b, j - 2))
                transpose(b)
                start_store(b, j)

        j = per_worker - 2
        wait_idx(1, j + 1)
        start_gather(1)
        wait_gather(0)
        wait_store(0, j - 2)
        transpose(0)
        start_store(0, j)
        wait_gather(1)
        wait_store(1, j - 1)
        transpose(1)
        start_store(1, j + 1)
        wait_store(0, j)
        wait_store(1, j + 1)

    return gather_kernel


def kernel(words, weight):
    nb4, ns_seq = words.shape
    nb = nb4 // BT
    w4 = (words.astype(jnp.int32)
          .reshape(nb, BT, ns_seq // DT, DT)
          .transpose(2, 0, 3, 1))
    out5 = _build(nb, ns_seq)(w4, weight)
    out = out5.transpose(2, 4, 0, 1, 3).reshape(nb4, ns_seq, EMBEDDING_DIM)
    return out
